# baseline (speedup 1.0000x reference)
import jax
import jax.numpy as jnp
from jax.experimental import pallas as pl
from jax.experimental.pallas import tpu as pltpu

_N = 10000
_NFEAT = 128
_NHID = 64
_NCLASS = 64

_BI = 400


def _prologue_kernel(x_ref, w1_ref, wh1_ref, wh2_ref, s1_ref, p_ref):
    x = x_ref[...]
    s1_ref[...] = jnp.dot(x, w1_ref[...] + wh1_ref[...],
                          preferred_element_type=jnp.float32)
    p_ref[...] = jnp.dot(x, wh2_ref[...], preferred_element_type=jnp.float32)


def _pass1_kernel(adj_ref, s1_ref, p_ref, b1_ref, w2_ref, out_ref):
    t = jnp.dot(adj_ref[...], s1_ref[...], preferred_element_type=jnp.float32)
    x1 = jnp.maximum(t + b1_ref[...], 0.0)
    out_ref[...] = (jnp.dot(x1, w2_ref[...], preferred_element_type=jnp.float32)
                    + p_ref[...])


def _pass2_kernel(adj_ref, s2_ref, b2_ref, out_ref):
    out_ref[...] = (jnp.dot(adj_ref[...], s2_ref[...],
                            preferred_element_type=jnp.float32)
                    + b2_ref[...])


def kernel(adj, x, W1, Wh1, b1, W2, Wh2, b2):
    grid = (_N // _BI,)
    b1_2d = b1.reshape(1, _NHID)
    b2_2d = b2.reshape(1, _NCLASS)

    s1, p = pl.pallas_call(
        _prologue_kernel,
        in_specs=[
            pl.BlockSpec((_N, _NFEAT), lambda: (0, 0)),
            pl.BlockSpec((_NFEAT, _NHID), lambda: (0, 0)),
            pl.BlockSpec((_NFEAT, _NHID), lambda: (0, 0)),
            pl.BlockSpec((_NFEAT, _NCLASS), lambda: (0, 0)),
        ],
        out_specs=[
            pl.BlockSpec((_N, _NHID), lambda: (0, 0)),
            pl.BlockSpec((_N, _NCLASS), lambda: (0, 0)),
        ],
        out_shape=[
            jax.ShapeDtypeStruct((_N, _NHID), jnp.float32),
            jax.ShapeDtypeStruct((_N, _NCLASS), jnp.float32),
        ],
    )(x, W1, Wh1, Wh2)

    s2 = pl.pallas_call(
        _pass1_kernel,
        grid=grid,
        in_specs=[
            pl.BlockSpec((_BI, _N), lambda i: (i, 0)),
            pl.BlockSpec((_N, _NHID), lambda i: (0, 0)),
            pl.BlockSpec((_BI, _NHID), lambda i: (i, 0)),
            pl.BlockSpec((1, _NHID), lambda i: (0, 0)),
            pl.BlockSpec((_NHID, _NCLASS), lambda i: (0, 0)),
        ],
        out_specs=pl.BlockSpec((_BI, _NCLASS), lambda i: (i, 0)),
        out_shape=jax.ShapeDtypeStruct((_N, _NCLASS), jnp.float32),
        compiler_params=pltpu.CompilerParams(
            dimension_semantics=("parallel",)),
    )(adj, s1, p, b1_2d, W2)

    out = pl.pallas_call(
        _pass2_kernel,
        grid=grid,
        in_specs=[
            pl.BlockSpec((_BI, _N), lambda i: (i, 0)),
            pl.BlockSpec((_N, _NCLASS), lambda i: (0, 0)),
            pl.BlockSpec((1, _NCLASS), lambda i: (0, 0)),
        ],
        out_specs=pl.BlockSpec((_BI, _NCLASS), lambda i: (i, 0)),
        out_shape=jax.ShapeDtypeStruct((_N, _NCLASS), jnp.float32),
        compiler_params=pltpu.CompilerParams(
            dimension_semantics=("parallel",)),
    )(adj, s2, b2_2d)

    return out


# speedup vs baseline: 1.0792x; 1.0792x over previous
import jax
import jax.numpy as jnp
from jax.experimental import pallas as pl
from jax.experimental.pallas import tpu as pltpu

_N = 10000
_NFEAT = 128
_NHID = 64
_NCLASS = 64

_BI = 400


def _gcn_kernel(adj_ref, x_ref, w1_ref, wh1_ref, wh2_ref, w2_ref, b1_ref,
                b2_ref, out_ref, s1_scr, p_scr, s2_scr):
    p = pl.program_id(0)
    i = pl.program_id(1)

    @pl.when((p == 0) & (i == 0))
    def _prologue():
        xx = x_ref[...]
        s1_scr[...] = jnp.dot(xx, w1_ref[...] + wh1_ref[...],
                              preferred_element_type=jnp.float32)
        p_scr[...] = jnp.dot(xx, wh2_ref[...],
                             preferred_element_type=jnp.float32)

    rows = pl.ds(pl.multiple_of(i * _BI, 8), _BI)

    @pl.when(p == 0)
    def _pass1():
        t = jnp.dot(adj_ref[...], s1_scr[...],
                    preferred_element_type=jnp.float32)
        x1 = jnp.maximum(t + b1_ref[...], 0.0)
        s2_blk = (jnp.dot(x1, w2_ref[...], preferred_element_type=jnp.float32)
                  + p_scr[rows, :])
        s2_scr[rows, :] = s2_blk
        out_ref[...] = s2_blk

    @pl.when(p == 1)
    def _pass2():
        out_ref[...] = (jnp.dot(adj_ref[...], s2_scr[...],
                                preferred_element_type=jnp.float32)
                        + b2_ref[...])


def kernel(adj, x, W1, Wh1, b1, W2, Wh2, b2):
    return pl.pallas_call(
        _gcn_kernel,
        grid=(2, _N // _BI),
        in_specs=[
            pl.BlockSpec((_BI, _N), lambda p, i: (i, 0)),
            pl.BlockSpec((_N, _NFEAT), lambda p, i: (0, 0)),
            pl.BlockSpec((_NFEAT, _NHID), lambda p, i: (0, 0)),
            pl.BlockSpec((_NFEAT, _NHID), lambda p, i: (0, 0)),
            pl.BlockSpec((_NFEAT, _NCLASS), lambda p, i: (0, 0)),
            pl.BlockSpec((_NHID, _NCLASS), lambda p, i: (0, 0)),
            pl.BlockSpec((1, _NHID), lambda p, i: (0, 0)),
            pl.BlockSpec((1, _NCLASS), lambda p, i: (0, 0)),
        ],
        out_specs=pl.BlockSpec((_BI, _NCLASS), lambda p, i: (p * i, 0)),
        out_shape=jax.ShapeDtypeStruct((_N, _NCLASS), jnp.float32),
        scratch_shapes=[
            pltpu.VMEM((_N, _NHID), jnp.float32),
            pltpu.VMEM((_N, _NCLASS), jnp.float32),
            pltpu.VMEM((_N, _NCLASS), jnp.float32),
        ],
        compiler_params=pltpu.CompilerParams(
            dimension_semantics=("arbitrary", "arbitrary")),
    )(adj, x, W1, Wh1, Wh2, W2, b1.reshape(1, _NHID), b2.reshape(1, _NCLASS))
